# Initial kernel scaffold; baseline (speedup 1.0000x reference)
import jax
import jax.numpy as jnp
from jax.experimental import pallas as pl


def kernel(x, edge_index, edge_attr, root_w, root_b, edge_w, edge_b):
    raise NotImplementedError("write your pallas kernel here")


# speedup vs baseline: 3.0946x; 3.0946x over previous
import functools

import jax
import jax.numpy as jnp
from jax import lax
from jax.experimental import pallas as pl
from jax.experimental.pallas import tpu as pltpu
from jax.experimental.pallas import tpu_sc as plsc

N_NODES = 50000
E = 400000
D = 16
NC, NS = 2, 16
NW = NC * NS
SL = 128
SLC = 14
CHUNK = SL * SLC
NCH = 7
PER_W = CHUNK * NCH
E_PAD = NW * PER_W
PAD_SRC_ROWS = 32
X_ROWS = N_NODES + PAD_SRC_ROWS
TPA = 3136
ACC_N = NS * TPA
ROWS_PER_W = PER_W // SL

def _gather_body(x_hbm, src_hbm, xsrc_hbm, idx_v, rows_v, sem):
    c = lax.axis_index("c")
    s = lax.axis_index("s")
    wid = s * NC + c

    def body(k, carry):
        row0 = wid * ROWS_PER_W + k * SLC
        pltpu.sync_copy(src_hbm.at[pl.ds(row0, SLC)], idx_v)
        descs = []
        for j in range(SLC):
            descs.append(pltpu.async_copy(
                x_hbm.at[idx_v.at[j]], rows_v.at[pl.ds(j * SL, SL)], sem))
        for dsc in descs:
            dsc.wait()
        pltpu.sync_copy(rows_v, xsrc_hbm.at[pl.ds(row0 * SL, CHUNK)])
        return carry

    lax.fori_loop(0, NCH, body, 0)


def _scatter_body(dst_hbm, msg_hbm, part_hbm, idx_v, rows_v, zbuf, acc):
    c = lax.axis_index("c")
    s = lax.axis_index("s")

    def zrow(i, carry):
        zbuf[i, :] = jnp.zeros((D,), jnp.float32)
        return carry

    lax.fori_loop(0, TPA, zrow, 0)
    pltpu.sync_copy(zbuf, acc.at[pl.ds(s * TPA, TPA)])
    plsc.subcore_barrier()

    wid = c * NS + s

    def body(k, carry):
        row0 = wid * ROWS_PER_W + k * SLC
        pltpu.sync_copy(dst_hbm.at[pl.ds(row0, SLC)], idx_v)
        pltpu.sync_copy(msg_hbm.at[pl.ds(row0 * SL, CHUNK)], rows_v)
        for j in range(SLC):
            pltpu.sync_copy(rows_v.at[pl.ds(j * SL, SL)],
                            acc.at[idx_v.at[j]], add=True)
        return carry

    lax.fori_loop(0, NCH, body, 0)

    plsc.subcore_barrier()
    pltpu.sync_copy(acc.at[pl.ds(s * TPA, TPA)],
                    part_hbm.at[c].at[pl.ds(s * TPA, TPA)])


TE = 2048


def _msg_body(xs_ref, ea_ref, r_ref, s_ref, w2_ref, b_ref, out_ref):
    xs = xs_ref[...]
    ea = ea_ref[...]
    xr = jnp.dot(xs, r_ref[...], preferred_element_type=jnp.float32)
    ar = jnp.dot(ea, s_ref[...], preferred_element_type=jnp.float32)
    t = xr * ar
    out_ref[...] = (jnp.dot(t, w2_ref[...], preferred_element_type=jnp.float32)
                    + jnp.dot(xs, b_ref[...], preferred_element_type=jnp.float32))


TN = 2000


def _comb_body(p_ref, x_ref, rwt_ref, rb_ref, out_ref):
    root = jnp.dot(x_ref[...], rwt_ref[...], preferred_element_type=jnp.float32)
    out_ref[...] = p_ref[0] + p_ref[1] + root + rb_ref[0:1, :]


def kernel(x, edge_index, edge_attr, root_w, root_b, edge_w, edge_b):
    x = x.astype(jnp.float32)
    src = edge_index[0].astype(jnp.int32)
    dst = edge_index[1].astype(jnp.int32)
    pad = E_PAD - E
    pad_ids = jnp.arange(pad, dtype=jnp.int32)
    src_p = jnp.concatenate([src, N_NODES + (pad_ids % PAD_SRC_ROWS)])
    dst_p = jnp.concatenate([dst, pad_ids % N_NODES])
    ea_p = jnp.concatenate([edge_attr, jnp.zeros((pad, D), jnp.float32)])
    x_aug = jnp.concatenate([x, jnp.zeros((PAD_SRC_ROWS, D), jnp.float32)])
    src2d = src_p.reshape(E_PAD // SL, SL)
    dst2d = dst_p.reshape(E_PAD // SL, SL)

    mesh = plsc.VectorSubcoreMesh(core_axis_name="c", subcore_axis_name="s",
                                  num_cores=NC, num_subcores=NS)
    sc_params = pltpu.CompilerParams(use_tc_tiling_on_sc=False)
    gather_kernel = pl.kernel(
        _gather_body,
        out_type=jax.ShapeDtypeStruct((E_PAD, D), jnp.float32),
        mesh=mesh,
        compiler_params=sc_params,
        scratch_types=[
            pltpu.VMEM((SLC, SL), jnp.int32),
            pltpu.VMEM((CHUNK, D), jnp.float32),
            pltpu.SemaphoreType.DMA,
        ],
    )
    scatter_kernel = pl.kernel(
        _scatter_body,
        out_type=jax.ShapeDtypeStruct((NC, ACC_N, D), jnp.float32),
        mesh=mesh,
        compiler_params=sc_params,
        scratch_types=[
            pltpu.VMEM((SLC, SL), jnp.int32),
            pltpu.VMEM((CHUNK, D), jnp.float32),
            pltpu.VMEM((TPA, D), jnp.float32),
            pltpu.VMEM_SHARED((ACC_N, D), jnp.float32),
        ],
    )

    x_src = gather_kernel(x_aug, src2d)

    eye = jnp.eye(D, dtype=jnp.float32)
    r_mat = jnp.kron(eye, jnp.ones((1, D), jnp.float32))
    s_mat = jnp.tile(eye, (1, D))
    w2 = edge_w.reshape(D, D, D).transpose(1, 2, 0).reshape(D * D, D)
    b_mat = edge_b.reshape(D, D).T

    msg = pl.pallas_call(
        _msg_body,
        grid=(E_PAD // TE,),
        in_specs=[
            pl.BlockSpec((TE, D), lambda i: (i, 0)),
            pl.BlockSpec((TE, D), lambda i: (i, 0)),
            pl.BlockSpec((D, D * D), lambda i: (0, 0)),
            pl.BlockSpec((D, D * D), lambda i: (0, 0)),
            pl.BlockSpec((D * D, D), lambda i: (0, 0)),
            pl.BlockSpec((D, D), lambda i: (0, 0)),
        ],
        out_specs=pl.BlockSpec((TE, D), lambda i: (i, 0)),
        out_shape=jax.ShapeDtypeStruct((E_PAD, D), jnp.float32),
    )(x_src, ea_p, r_mat, s_mat, w2, b_mat)

    parts = scatter_kernel(dst2d, msg)

    rwt = root_w.T
    rbb = jnp.broadcast_to(root_b, (8, D))
    out = pl.pallas_call(
        _comb_body,
        grid=(N_NODES // TN,),
        in_specs=[
            pl.BlockSpec((NC, TN, D), lambda i: (0, i, 0)),
            pl.BlockSpec((TN, D), lambda i: (i, 0)),
            pl.BlockSpec((D, D), lambda i: (0, 0)),
            pl.BlockSpec((8, D), lambda i: (0, 0)),
        ],
        out_specs=pl.BlockSpec((TN, D), lambda i: (i, 0)),
        out_shape=jax.ShapeDtypeStruct((N_NODES, D), jnp.float32),
    )(parts, x, rwt, rbb)
    return out


# speedup vs baseline: 3.5760x; 1.1555x over previous
import functools

import jax
import jax.numpy as jnp
from jax import lax
from jax.experimental import pallas as pl
from jax.experimental.pallas import tpu as pltpu
from jax.experimental.pallas import tpu_sc as plsc

N_NODES = 50000
E = 400000
D = 16
NC, NS = 2, 16
NW = NC * NS
SL = 128
SLC = 14
CHUNK = SL * SLC
NCH = 7
PER_W = CHUNK * NCH
E_PAD = NW * PER_W
TPA = 3136
ACC_N = NS * TPA
ROWS_PER_W = PER_W // SL

def _gather_body(x_hbm, src_hbm, xsrc_hbm, idx_v, rows_v, sem):
    c = lax.axis_index("c")
    s = lax.axis_index("s")
    wid = s * NC + c

    def body(k, carry):
        row0 = wid * ROWS_PER_W + k * SLC
        pltpu.sync_copy(src_hbm.at[pl.ds(row0, SLC)], idx_v)
        descs = []
        for j in range(SLC):
            descs.append(pltpu.async_copy(
                x_hbm.at[idx_v.at[j]], rows_v.at[pl.ds(j * SL, SL)], sem))
        for dsc in descs:
            dsc.wait()
        pltpu.sync_copy(rows_v, xsrc_hbm.at[pl.ds(row0 * SL, CHUNK)])
        return carry

    lax.fori_loop(0, NCH, body, 0)


def _scatter_body(dst_hbm, msg_hbm, part_hbm, idx_v, rows_v, zbuf, acc):
    c = lax.axis_index("c")
    s = lax.axis_index("s")

    def zrow(i, carry):
        zbuf[i, :] = jnp.zeros((D,), jnp.float32)
        return carry

    lax.fori_loop(0, TPA, zrow, 0)
    pltpu.sync_copy(zbuf, acc.at[pl.ds(s * TPA, TPA)])
    plsc.subcore_barrier()

    wid = c * NS + s

    def body(k, carry):
        row0 = wid * ROWS_PER_W + k * SLC
        pltpu.sync_copy(dst_hbm.at[pl.ds(row0, SLC)], idx_v)
        pltpu.sync_copy(msg_hbm.at[pl.ds(row0 * SL, CHUNK)], rows_v)
        for j in range(SLC):
            pltpu.sync_copy(rows_v.at[pl.ds(j * SL, SL)],
                            acc.at[idx_v.at[j]], add=True)
        return carry

    lax.fori_loop(0, NCH, body, 0)

    plsc.subcore_barrier()
    pltpu.sync_copy(acc.at[pl.ds(s * TPA, TPA)],
                    part_hbm.at[c].at[pl.ds(s * TPA, TPA)])


TE = 4096


def _msg_body(xs_ref, ea_ref, r_ref, s_ref, w2_ref, b_ref, out_ref):
    xs = xs_ref[...].astype(jnp.bfloat16)
    ea = ea_ref[...].astype(jnp.bfloat16)
    xr = jnp.dot(xs, r_ref[...].astype(jnp.bfloat16),
                 preferred_element_type=jnp.float32)
    ar = jnp.dot(ea, s_ref[...].astype(jnp.bfloat16),
                 preferred_element_type=jnp.float32)
    t = (xr * ar).astype(jnp.bfloat16)
    out_ref[...] = (jnp.dot(t, w2_ref[...].astype(jnp.bfloat16),
                            preferred_element_type=jnp.float32)
                    + jnp.dot(xs, b_ref[...].astype(jnp.bfloat16),
                              preferred_element_type=jnp.float32))


TN = 2000


def _comb_body(p_ref, x_ref, rwt_ref, rb_ref, out_ref):
    root = jnp.dot(x_ref[...], rwt_ref[...], preferred_element_type=jnp.float32)
    out_ref[...] = p_ref[0] + p_ref[1] + root + rb_ref[0:1, :]


def kernel(x, edge_index, edge_attr, root_w, root_b, edge_w, edge_b):
    x = x.astype(jnp.float32)
    src = edge_index[0].astype(jnp.int32)
    dst = edge_index[1].astype(jnp.int32)
    pad = E_PAD - E
    pad_ids = jnp.arange(pad, dtype=jnp.int32)
    src_p = jnp.concatenate([src, pad_ids % N_NODES])
    dst_p = jnp.concatenate([dst, N_NODES + pad_ids % (ACC_N - N_NODES)])
    src2d = src_p.reshape(E_PAD // SL, SL)
    dst2d = dst_p.reshape(E_PAD // SL, SL)

    mesh = plsc.VectorSubcoreMesh(core_axis_name="c", subcore_axis_name="s",
                                  num_cores=NC, num_subcores=NS)
    sc_params = pltpu.CompilerParams(use_tc_tiling_on_sc=False)
    gather_kernel = pl.kernel(
        _gather_body,
        out_type=jax.ShapeDtypeStruct((E_PAD, D), jnp.float32),
        mesh=mesh,
        compiler_params=sc_params,
        scratch_types=[
            pltpu.VMEM((SLC, SL), jnp.int32),
            pltpu.VMEM((CHUNK, D), jnp.float32),
            pltpu.SemaphoreType.DMA,
        ],
    )
    scatter_kernel = pl.kernel(
        _scatter_body,
        out_type=jax.ShapeDtypeStruct((NC, ACC_N, D), jnp.float32),
        mesh=mesh,
        compiler_params=sc_params,
        scratch_types=[
            pltpu.VMEM((SLC, SL), jnp.int32),
            pltpu.VMEM((CHUNK, D), jnp.float32),
            pltpu.VMEM((TPA, D), jnp.float32),
            pltpu.VMEM_SHARED((ACC_N, D), jnp.float32),
        ],
    )

    x_src = gather_kernel(x, src2d)

    eye = jnp.eye(D, dtype=jnp.float32)
    r_mat = jnp.kron(eye, jnp.ones((1, D), jnp.float32))
    s_mat = jnp.tile(eye, (1, D))
    w2 = edge_w.reshape(D, D, D).transpose(1, 2, 0).reshape(D * D, D)
    b_mat = edge_b.reshape(D, D).T

    msg = pl.pallas_call(
        _msg_body,
        grid=(E_PAD // TE,),
        in_specs=[
            pl.BlockSpec((TE, D), lambda i: (i, 0)),
            pl.BlockSpec((TE, D), lambda i: (i, 0)),
            pl.BlockSpec((D, D * D), lambda i: (0, 0)),
            pl.BlockSpec((D, D * D), lambda i: (0, 0)),
            pl.BlockSpec((D * D, D), lambda i: (0, 0)),
            pl.BlockSpec((D, D), lambda i: (0, 0)),
        ],
        out_specs=pl.BlockSpec((TE, D), lambda i: (i, 0)),
        out_shape=jax.ShapeDtypeStruct((E_PAD, D), jnp.float32),
    )(x_src, edge_attr, r_mat, s_mat, w2, b_mat)

    parts = scatter_kernel(dst2d, msg)

    rwt = root_w.T
    rbb = jnp.broadcast_to(root_b, (8, D))
    out = pl.pallas_call(
        _comb_body,
        grid=(N_NODES // TN,),
        in_specs=[
            pl.BlockSpec((NC, TN, D), lambda i: (0, i, 0)),
            pl.BlockSpec((TN, D), lambda i: (i, 0)),
            pl.BlockSpec((D, D), lambda i: (0, 0)),
            pl.BlockSpec((8, D), lambda i: (0, 0)),
        ],
        out_specs=pl.BlockSpec((TN, D), lambda i: (i, 0)),
        out_shape=jax.ShapeDtypeStruct((N_NODES, D), jnp.float32),
    )(parts, x, rwt, rbb)
    return out


# speedup vs baseline: 5.5656x; 1.5564x over previous
import functools

import jax
import jax.numpy as jnp
from jax import lax
from jax.experimental import pallas as pl
from jax.experimental.pallas import tpu as pltpu
from jax.experimental.pallas import tpu_sc as plsc

N_NODES = 50000
E = 400000
D = 16
NC, NS = 2, 16
NW = NC * NS
SL = 128
SLC = 14
CHUNK = SL * SLC
NCH = 7
PER_W = CHUNK * NCH
E_PAD = NW * PER_W
TPA = 3136
ACC_N = NS * TPA
ROWS_PER_W = PER_W // SL

def _gather_body(x_hbm, src_hbm, xsrc_hbm, idx_v, rows_v, sem):
    c = lax.axis_index("c")
    s = lax.axis_index("s")
    wid = s * NC + c

    def body(k, carry):
        row0 = wid * ROWS_PER_W + k * SLC
        pltpu.sync_copy(src_hbm.at[pl.ds(row0, SLC)], idx_v)
        descs = []
        for j in range(SLC):
            descs.append(pltpu.async_copy(
                x_hbm.at[idx_v.at[j]], rows_v.at[pl.ds(j * SL, SL)], sem))
        for dsc in descs:
            dsc.wait()
        pltpu.sync_copy(rows_v, xsrc_hbm.at[pl.ds(row0 * SL, CHUNK)])
        return carry

    lax.fori_loop(0, NCH, body, 0)


def _scatter_body(dst_hbm, msg_hbm, part_hbm, idx_v, rows_v, zbuf, acc):
    c = lax.axis_index("c")
    s = lax.axis_index("s")

    def zrow(i, carry):
        zbuf[i, :] = jnp.zeros((D,), jnp.float32)
        return carry

    lax.fori_loop(0, TPA, zrow, 0)
    pltpu.sync_copy(zbuf, acc.at[pl.ds(s * TPA, TPA)])
    plsc.subcore_barrier()

    wid = c * NS + s

    def body(k, carry):
        row0 = wid * ROWS_PER_W + k * SLC
        pltpu.sync_copy(dst_hbm.at[pl.ds(row0, SLC)], idx_v)
        pltpu.sync_copy(msg_hbm.at[pl.ds(row0 * SL, CHUNK)], rows_v)
        for j in range(SLC):
            pltpu.sync_copy(rows_v.at[pl.ds(j * SL, SL)],
                            acc.at[idx_v.at[j]], add=True)
        return carry

    lax.fori_loop(0, NCH, body, 0)

    plsc.subcore_barrier()
    pltpu.sync_copy(acc.at[pl.ds(s * TPA, TPA)],
                    part_hbm.at[c].at[pl.ds(s * TPA, TPA)])


TEW = 512


def _msg_body(xw_ref, ea_ref, r_ref, s_ref, w2_ref, b_ref, out_ref):
    xw = xw_ref[...].astype(jnp.bfloat16)
    ea = ea_ref[...].astype(jnp.bfloat16)
    xr = jnp.dot(xw, r_ref[...], preferred_element_type=jnp.float32)
    ar = jnp.dot(ea, s_ref[...], preferred_element_type=jnp.float32)
    t = (xr * ar).astype(jnp.bfloat16)
    out_ref[...] = (jnp.dot(t, w2_ref[...], preferred_element_type=jnp.float32)
                    + jnp.dot(xw, b_ref[...], preferred_element_type=jnp.float32))


TNW = 392


def _comb_body(p_ref, x_ref, rwt_ref, rb_ref, out_ref):
    root = jnp.dot(x_ref[...].astype(jnp.bfloat16), rwt_ref[...],
                   preferred_element_type=jnp.float32)
    out_ref[...] = p_ref[0] + p_ref[1] + root + rb_ref[0:1, :]


def kernel(x, edge_index, edge_attr, root_w, root_b, edge_w, edge_b):
    x = x.astype(jnp.float32)
    src = edge_index[0].astype(jnp.int32)
    dst = edge_index[1].astype(jnp.int32)
    pad = E_PAD - E
    pad_ids = jnp.arange(pad, dtype=jnp.int32)
    src_p = jnp.concatenate([src, pad_ids % N_NODES])
    dst_p = jnp.concatenate([dst, N_NODES + pad_ids % (ACC_N - N_NODES)])
    src2d = src_p.reshape(E_PAD // SL, SL)
    dst2d = dst_p.reshape(E_PAD // SL, SL)

    mesh = plsc.VectorSubcoreMesh(core_axis_name="c", subcore_axis_name="s",
                                  num_cores=NC, num_subcores=NS)
    sc_params = pltpu.CompilerParams(use_tc_tiling_on_sc=False)
    gather_kernel = pl.kernel(
        _gather_body,
        out_type=jax.ShapeDtypeStruct((E_PAD, D), jnp.float32),
        mesh=mesh,
        compiler_params=sc_params,
        scratch_types=[
            pltpu.VMEM((SLC, SL), jnp.int32),
            pltpu.VMEM((CHUNK, D), jnp.float32),
            pltpu.SemaphoreType.DMA,
        ],
    )
    scatter_kernel = pl.kernel(
        _scatter_body,
        out_type=jax.ShapeDtypeStruct((NC, ACC_N, D), jnp.float32),
        mesh=mesh,
        compiler_params=sc_params,
        scratch_types=[
            pltpu.VMEM((SLC, SL), jnp.int32),
            pltpu.VMEM((CHUNK, D), jnp.float32),
            pltpu.VMEM((TPA, D), jnp.float32),
            pltpu.VMEM_SHARED((ACC_N, D), jnp.float32),
        ],
    )

    x_src = gather_kernel(x, src2d)
    xw = x_src.reshape(E_PAD // 8, 128)
    eaw = edge_attr.reshape(E // 8, 128)

    eye = jnp.eye(D, dtype=jnp.float32)
    eye8 = jnp.eye(8, dtype=jnp.float32)
    r_mat = jnp.kron(eye, jnp.ones((1, D), jnp.float32))
    s_mat = jnp.tile(eye, (1, D))
    w2 = edge_w.reshape(D, D, D).transpose(1, 2, 0).reshape(D * D, D)
    b_mat = edge_b.reshape(D, D).T
    bd_r = jnp.kron(eye8, r_mat).astype(jnp.bfloat16)
    bd_s = jnp.kron(eye8, s_mat).astype(jnp.bfloat16)
    bd_w2 = jnp.kron(eye8, w2).astype(jnp.bfloat16)
    bd_b = jnp.kron(eye8, b_mat).astype(jnp.bfloat16)

    msg_w = pl.pallas_call(
        _msg_body,
        grid=(E_PAD // 8 // TEW,),
        in_specs=[
            pl.BlockSpec((TEW, 128), lambda i: (i, 0)),
            pl.BlockSpec((TEW, 128), lambda i: (i, 0)),
            pl.BlockSpec((128, 2048), lambda i: (0, 0)),
            pl.BlockSpec((128, 2048), lambda i: (0, 0)),
            pl.BlockSpec((2048, 128), lambda i: (0, 0)),
            pl.BlockSpec((128, 128), lambda i: (0, 0)),
        ],
        out_specs=pl.BlockSpec((TEW, 128), lambda i: (i, 0)),
        out_shape=jax.ShapeDtypeStruct((E_PAD // 8, 128), jnp.float32),
    )(xw, eaw, bd_r, bd_s, bd_w2, bd_b)

    parts = scatter_kernel(dst2d, msg_w.reshape(E_PAD, D))

    parts_w = parts.reshape(NC, ACC_N * D // 128, 128)
    xw8 = x.reshape(N_NODES * D // 128, 128)
    bd_rwt = jnp.kron(eye8, root_w.T).astype(jnp.bfloat16)
    rbw = jnp.broadcast_to(jnp.tile(root_b, 8), (8, 128))
    acc_w = ACC_N * D // 128
    out_w = pl.pallas_call(
        _comb_body,
        grid=(acc_w // TNW,),
        in_specs=[
            pl.BlockSpec((NC, TNW, 128), lambda i: (0, i, 0)),
            pl.BlockSpec((TNW, 128), lambda i: (i, 0)),
            pl.BlockSpec((128, 128), lambda i: (0, 0)),
            pl.BlockSpec((8, 128), lambda i: (0, 0)),
        ],
        out_specs=pl.BlockSpec((TNW, 128), lambda i: (i, 0)),
        out_shape=jax.ShapeDtypeStruct((acc_w, 128), jnp.float32),
    )(parts_w, xw8, bd_rwt, rbw)
    return out_w.reshape(ACC_N, D)[:N_NODES]


# speedup vs baseline: 7.0280x; 1.2627x over previous
import functools

import jax
import jax.numpy as jnp
from jax import lax
from jax.experimental import pallas as pl
from jax.experimental.pallas import tpu as pltpu
from jax.experimental.pallas import tpu_sc as plsc

N_NODES = 50000
E = 400000
D = 16
NC, NS = 2, 16
NW = NC * NS
SL = 128
SLC = 14
CHUNK = SL * SLC
NCH = 7
PER_W = CHUNK * NCH
E_PAD = NW * PER_W
TPA = 3136
ACC_N = NS * TPA
ROWS_PER_W = PER_W // SL

def _gather_body(x_hbm, src_hbm, xsrc_hbm, idx_v, rows_v, sem):
    c = lax.axis_index("c")
    s = lax.axis_index("s")
    wid = s * NC + c

    def body(k, carry):
        row0 = wid * ROWS_PER_W + k * SLC
        pltpu.sync_copy(src_hbm.at[pl.ds(row0, SLC)], idx_v)
        descs = []
        for j in range(SLC):
            descs.append(pltpu.async_copy(
                x_hbm.at[idx_v.at[j]], rows_v.at[pl.ds(j * SL, SL)], sem))
        for dsc in descs:
            dsc.wait()
        pltpu.sync_copy(rows_v, xsrc_hbm.at[pl.ds(row0 * SL, CHUNK)])
        return carry

    lax.fori_loop(0, NCH, body, 0)


def _scatter_body(dst_hbm, msg_hbm, part_hbm, idx_v, rows_v, zbuf, acc):
    c = lax.axis_index("c")
    s = lax.axis_index("s")

    def zrow(i, carry):
        zbuf[i, :] = jnp.zeros((D,), jnp.float32)
        return carry

    lax.fori_loop(0, TPA, zrow, 0)
    pltpu.sync_copy(zbuf, acc.at[pl.ds(s * TPA, TPA)])
    plsc.subcore_barrier()

    wid = c * NS + s

    def body(k, carry):
        row0 = wid * ROWS_PER_W + k * SLC
        pltpu.sync_copy(dst_hbm.at[pl.ds(row0, SLC)], idx_v)
        pltpu.sync_copy(msg_hbm.at[pl.ds(row0 * SL, CHUNK)], rows_v)
        for j in range(SLC):
            pltpu.sync_copy(rows_v.at[pl.ds(j * SL, SL)],
                            acc.at[idx_v.at[j]], add=True)
        return carry

    lax.fori_loop(0, NCH, body, 0)

    plsc.subcore_barrier()
    pltpu.sync_copy(acc.at[pl.ds(s * TPA, TPA)],
                    part_hbm.at[c].at[pl.ds(s * TPA, TPA)])


TEW = 512


def _msg_body(xw_ref, eat_ref, r_ref, s_ref, w2_ref, b_ref, out_ref):
    xw = xw_ref[...].astype(jnp.bfloat16)
    eat = eat_ref[...].astype(jnp.bfloat16)
    ea_n = jax.lax.dot_general(eat, s_ref[...], (((0,), (0,)), ((), ())),
                               preferred_element_type=jnp.float32)
    ar = ea_n.reshape(TEW, 2048)
    xr = jnp.dot(xw, r_ref[...], preferred_element_type=jnp.float32)
    t = (xr * ar).astype(jnp.bfloat16)
    out_ref[...] = (jnp.dot(t, w2_ref[...], preferred_element_type=jnp.float32)
                    + jnp.dot(xw, b_ref[...], preferred_element_type=jnp.float32))


TNW = 392


def _comb_body(p_ref, x_ref, rwt_ref, rb_ref, out_ref):
    root = jnp.dot(x_ref[...].astype(jnp.bfloat16), rwt_ref[...],
                   preferred_element_type=jnp.float32)
    out_ref[...] = p_ref[0] + p_ref[1] + root + rb_ref[0:1, :]


def kernel(x, edge_index, edge_attr, root_w, root_b, edge_w, edge_b):
    x = x.astype(jnp.float32)
    src = edge_index[0].astype(jnp.int32)
    dst = edge_index[1].astype(jnp.int32)
    pad = E_PAD - E
    pad_ids = jnp.arange(pad, dtype=jnp.int32)
    src_p = jnp.concatenate([src, pad_ids % N_NODES])
    dst_p = jnp.concatenate([dst, N_NODES + pad_ids % (ACC_N - N_NODES)])
    src2d = src_p.reshape(E_PAD // SL, SL)
    dst2d = dst_p.reshape(E_PAD // SL, SL)

    mesh = plsc.VectorSubcoreMesh(core_axis_name="c", subcore_axis_name="s",
                                  num_cores=NC, num_subcores=NS)
    sc_params = pltpu.CompilerParams(use_tc_tiling_on_sc=False)
    gather_kernel = pl.kernel(
        _gather_body,
        out_type=jax.ShapeDtypeStruct((E_PAD, D), jnp.float32),
        mesh=mesh,
        compiler_params=sc_params,
        scratch_types=[
            pltpu.VMEM((SLC, SL), jnp.int32),
            pltpu.VMEM((CHUNK, D), jnp.float32),
            pltpu.SemaphoreType.DMA,
        ],
    )
    scatter_kernel = pl.kernel(
        _scatter_body,
        out_type=jax.ShapeDtypeStruct((NC, ACC_N, D), jnp.float32),
        mesh=mesh,
        compiler_params=sc_params,
        scratch_types=[
            pltpu.VMEM((SLC, SL), jnp.int32),
            pltpu.VMEM((CHUNK, D), jnp.float32),
            pltpu.VMEM((TPA, D), jnp.float32),
            pltpu.VMEM_SHARED((ACC_N, D), jnp.float32),
        ],
    )

    x_src = gather_kernel(x, src2d)
    xw = x_src.reshape(E_PAD // 8, 128)
    eat = edge_attr.T

    eye = jnp.eye(D, dtype=jnp.float32)
    eye8 = jnp.eye(8, dtype=jnp.float32)
    r_mat = jnp.kron(eye, jnp.ones((1, D), jnp.float32))
    s_mat = jnp.tile(eye, (1, D))
    w2 = edge_w.reshape(D, D, D).transpose(1, 2, 0).reshape(D * D, D)
    b_mat = edge_b.reshape(D, D).T
    bd_r = jnp.kron(eye8, r_mat).astype(jnp.bfloat16)
    bd_w2 = jnp.kron(eye8, w2).astype(jnp.bfloat16)
    bd_b = jnp.kron(eye8, b_mat).astype(jnp.bfloat16)

    msg_w = pl.pallas_call(
        _msg_body,
        grid=(E_PAD // 8 // TEW,),
        in_specs=[
            pl.BlockSpec((TEW, 128), lambda i: (i, 0)),
            pl.BlockSpec((D, TEW * 8), lambda i: (0, i)),
            pl.BlockSpec((128, 2048), lambda i: (0, 0)),
            pl.BlockSpec((D, D * D), lambda i: (0, 0)),
            pl.BlockSpec((2048, 128), lambda i: (0, 0)),
            pl.BlockSpec((128, 128), lambda i: (0, 0)),
        ],
        out_specs=pl.BlockSpec((TEW, 128), lambda i: (i, 0)),
        out_shape=jax.ShapeDtypeStruct((E_PAD // 8, 128), jnp.float32),
    )(xw, eat, bd_r, s_mat.astype(jnp.bfloat16), bd_w2, bd_b)

    parts = scatter_kernel(dst2d, msg_w.reshape(E_PAD, D))

    parts_w = parts.reshape(NC, ACC_N * D // 128, 128)
    xw8 = x.reshape(N_NODES * D // 128, 128)
    bd_rwt = jnp.kron(eye8, root_w.T).astype(jnp.bfloat16)
    rbw = jnp.broadcast_to(jnp.tile(root_b, 8), (8, 128))
    acc_w = ACC_N * D // 128
    out_w = pl.pallas_call(
        _comb_body,
        grid=(acc_w // TNW,),
        in_specs=[
            pl.BlockSpec((NC, TNW, 128), lambda i: (0, i, 0)),
            pl.BlockSpec((TNW, 128), lambda i: (i, 0)),
            pl.BlockSpec((128, 128), lambda i: (0, 0)),
            pl.BlockSpec((8, 128), lambda i: (0, 0)),
        ],
        out_specs=pl.BlockSpec((TNW, 128), lambda i: (i, 0)),
        out_shape=jax.ShapeDtypeStruct((acc_w, 128), jnp.float32),
    )(parts_w, xw8, bd_rwt, rbw)
    return out_w.reshape(ACC_N, D)[:N_NODES]


# speedup vs baseline: 7.3756x; 1.0495x over previous
import jax
import jax.numpy as jnp
from jax import lax
from jax.experimental import pallas as pl
from jax.experimental.pallas import tpu as pltpu
from jax.experimental.pallas import tpu_sc as plsc

N_NODES = 50000
E = 400000
D = 16
NC, NS = 2, 16
NW = NC * NS
SL = 128
SLC = 7
CHUNK = SL * SLC
NCH = 7
PER_W = CHUNK * NCH
E_HALF = NW * PER_W
E_PAD = 2 * E_HALF
TPA = 3136
ACC_N = NS * TPA
ROWS_PER_W = PER_W // SL
HALF_ROWS = E_HALF // SL


def _gather_body(half, x_hbm, src_hbm, xsrc_hbm, idx_v, rows_v, sem):
    c = lax.axis_index("c")
    s = lax.axis_index("s")
    wid = s * NC + c

    def body(k, carry):
        rel = wid * ROWS_PER_W + k * SLC
        pltpu.sync_copy(src_hbm.at[pl.ds(half * HALF_ROWS + rel, SLC)], idx_v)
        descs = []
        for j in range(SLC):
            descs.append(pltpu.async_copy(
                x_hbm.at[idx_v.at[j]], rows_v.at[pl.ds(j * SL, SL)], sem))
        for dsc in descs:
            dsc.wait()
        pltpu.sync_copy(rows_v, xsrc_hbm.at[pl.ds(rel * SL, CHUNK)])
        return carry

    lax.fori_loop(0, NCH, body, 0)


def _scatter_body(half, dst_hbm, msg_hbm, part_hbm, idx_v, rows_v, zbuf, acc):
    c = lax.axis_index("c")
    s = lax.axis_index("s")

    def zrow(i, carry):
        zbuf[i, :] = jnp.zeros((D,), jnp.float32)
        return carry

    lax.fori_loop(0, TPA, zrow, 0)
    pltpu.sync_copy(zbuf, acc.at[pl.ds(s * TPA, TPA)])
    plsc.subcore_barrier()

    wid = c * NS + s

    def body(k, carry):
        rel = wid * ROWS_PER_W + k * SLC
        pltpu.sync_copy(dst_hbm.at[pl.ds(half * HALF_ROWS + rel, SLC)], idx_v)
        pltpu.sync_copy(msg_hbm.at[pl.ds(rel * SL, CHUNK)], rows_v)
        for j in range(SLC):
            pltpu.sync_copy(rows_v.at[pl.ds(j * SL, SL)],
                            acc.at[idx_v.at[j]], add=True)
        return carry

    lax.fori_loop(0, NCH, body, 0)

    plsc.subcore_barrier()
    pltpu.sync_copy(acc.at[pl.ds(s * TPA, TPA)],
                    part_hbm.at[c].at[pl.ds(s * TPA, TPA)])


TEW = 512


def _msg_body(xw_ref, eat_ref, r_ref, s_ref, w2_ref, b_ref, out_ref):
    xw = xw_ref[...].astype(jnp.bfloat16)
    eat = eat_ref[...].astype(jnp.bfloat16)
    ea_n = jax.lax.dot_general(eat, s_ref[...], (((0,), (0,)), ((), ())),
                               preferred_element_type=jnp.float32)
    ar = ea_n.reshape(TEW, 2048)
    xr = jnp.dot(xw, r_ref[...], preferred_element_type=jnp.float32)
    t = (xr * ar).astype(jnp.bfloat16)
    out_ref[...] = (jnp.dot(t, w2_ref[...], preferred_element_type=jnp.float32)
                    + jnp.dot(xw, b_ref[...], preferred_element_type=jnp.float32))


TNW = 392


def _comb_body(pa_ref, pb_ref, x_ref, rwt_ref, rb_ref, out_ref):
    root = jnp.dot(x_ref[...].astype(jnp.bfloat16), rwt_ref[...],
                   preferred_element_type=jnp.float32)
    out_ref[...] = (pa_ref[0] + pa_ref[1] + pb_ref[0] + pb_ref[1]
                    + root + rb_ref[0:1, :])


def kernel(x, edge_index, edge_attr, root_w, root_b, edge_w, edge_b):
    x = x.astype(jnp.float32)
    src = edge_index[0].astype(jnp.int32)
    dst = edge_index[1].astype(jnp.int32)
    pad = E_PAD - E
    pad_ids = jnp.arange(pad, dtype=jnp.int32)
    src_p = jnp.concatenate([src, pad_ids % N_NODES])
    dst_p = jnp.concatenate([dst, N_NODES + pad_ids % (ACC_N - N_NODES)])
    src2d = src_p.reshape(E_PAD // SL, SL)
    dst2d = dst_p.reshape(E_PAD // SL, SL)

    mesh = plsc.VectorSubcoreMesh(core_axis_name="c", subcore_axis_name="s",
                                  num_cores=NC, num_subcores=NS)
    sc_params = pltpu.CompilerParams(use_tc_tiling_on_sc=False)

    def make_gather(half):
        return pl.kernel(
            lambda *a: _gather_body(half, *a),
            out_type=jax.ShapeDtypeStruct((E_HALF, D), jnp.float32),
            mesh=mesh,
            compiler_params=sc_params,
            scratch_types=[
                pltpu.VMEM((SLC, SL), jnp.int32),
                pltpu.VMEM((CHUNK, D), jnp.float32),
                pltpu.SemaphoreType.DMA,
            ],
        )

    def make_scatter(half):
        return pl.kernel(
            lambda *a: _scatter_body(half, *a),
            out_type=jax.ShapeDtypeStruct((NC, ACC_N, D), jnp.float32),
            mesh=mesh,
            compiler_params=sc_params,
            scratch_types=[
                pltpu.VMEM((SLC, SL), jnp.int32),
                pltpu.VMEM((CHUNK, D), jnp.float32),
                pltpu.VMEM((TPA, D), jnp.float32),
                pltpu.VMEM_SHARED((ACC_N, D), jnp.float32),
            ],
        )

    eat = edge_attr.T

    eye = jnp.eye(D, dtype=jnp.float32)
    eye8 = jnp.eye(8, dtype=jnp.float32)
    r_mat = jnp.kron(eye, jnp.ones((1, D), jnp.float32))
    s_mat = jnp.tile(eye, (1, D))
    w2 = edge_w.reshape(D, D, D).transpose(1, 2, 0).reshape(D * D, D)
    b_mat = edge_b.reshape(D, D).T
    bd_r = jnp.kron(eye8, r_mat).astype(jnp.bfloat16)
    bd_w2 = jnp.kron(eye8, w2).astype(jnp.bfloat16)
    bd_b = jnp.kron(eye8, b_mat).astype(jnp.bfloat16)
    s_bf = s_mat.astype(jnp.bfloat16)

    def msg_half(xw_h, half):
        base = half * (E_HALF // 8 // TEW)
        return pl.pallas_call(
            _msg_body,
            grid=(E_HALF // 8 // TEW,),
            in_specs=[
                pl.BlockSpec((TEW, 128), lambda i: (i, 0)),
                pl.BlockSpec((D, TEW * 8), lambda i, b=base: (0, i + b)),
                pl.BlockSpec((128, 2048), lambda i: (0, 0)),
                pl.BlockSpec((D, D * D), lambda i: (0, 0)),
                pl.BlockSpec((2048, 128), lambda i: (0, 0)),
                pl.BlockSpec((128, 128), lambda i: (0, 0)),
            ],
            out_specs=pl.BlockSpec((TEW, 128), lambda i: (i, 0)),
            out_shape=jax.ShapeDtypeStruct((E_HALF // 8, 128), jnp.float32),
        )(xw_h, eat, bd_r, s_bf, bd_w2, bd_b)

    x_src0 = make_gather(0)(x, src2d)
    x_src1 = make_gather(1)(x, src2d)
    msg0 = msg_half(x_src0.reshape(E_HALF // 8, 128), 0)
    msg1 = msg_half(x_src1.reshape(E_HALF // 8, 128), 1)
    parts0 = make_scatter(0)(dst2d, msg0.reshape(E_HALF, D))
    parts1 = make_scatter(1)(dst2d, msg1.reshape(E_HALF, D))

    parts0_w = parts0.reshape(NC, ACC_N * D // 128, 128)
    parts1_w = parts1.reshape(NC, ACC_N * D // 128, 128)
    xw8 = x.reshape(N_NODES * D // 128, 128)
    bd_rwt = jnp.kron(eye8, root_w.T).astype(jnp.bfloat16)
    rbw = jnp.broadcast_to(jnp.tile(root_b, 8), (8, 128))
    acc_w = ACC_N * D // 128
    out_w = pl.pallas_call(
        _comb_body,
        grid=(acc_w // TNW,),
        in_specs=[
            pl.BlockSpec((NC, TNW, 128), lambda i: (0, i, 0)),
            pl.BlockSpec((NC, TNW, 128), lambda i: (0, i, 0)),
            pl.BlockSpec((TNW, 128), lambda i: (i, 0)),
            pl.BlockSpec((128, 128), lambda i: (0, 0)),
            pl.BlockSpec((8, 128), lambda i: (0, 0)),
        ],
        out_specs=pl.BlockSpec((TNW, 128), lambda i: (i, 0)),
        out_shape=jax.ShapeDtypeStruct((acc_w, 128), jnp.float32),
    )(parts0_w, parts1_w, xw8, bd_rwt, rbw)
    return out_w.reshape(ACC_N, D)[:N_NODES]


# speedup vs baseline: 7.5552x; 1.0244x over previous
import jax
import jax.numpy as jnp
from jax import lax
from jax.experimental import pallas as pl
from jax.experimental.pallas import tpu as pltpu
from jax.experimental.pallas import tpu_sc as plsc

N_NODES = 50000
E = 400000
D = 16
NC, NS = 2, 16
NW = NC * NS
SL = 128
SLC = 7
CHUNK = SL * SLC
NCH = 7
PER_W = CHUNK * NCH
E_HALF = NW * PER_W
E_PAD = 2 * E_HALF
TPA = 3136
ACC_N = NS * TPA
ROWS_PER_W = PER_W // SL
HALF_ROWS = E_HALF // SL


def _gather_body(half, x_hbm, src_hbm, xsrc_hbm, idx_v, rows_v, sem):
    c = lax.axis_index("c")
    s = lax.axis_index("s")
    wid = s * NC + c

    def body(k, carry):
        rel = wid * ROWS_PER_W + k * SLC
        pltpu.sync_copy(src_hbm.at[pl.ds(half * HALF_ROWS + rel, SLC)], idx_v)
        descs = []
        for j in range(SLC):
            descs.append(pltpu.async_copy(
                x_hbm.at[idx_v.at[j]], rows_v.at[pl.ds(j * SL, SL)], sem))
        for dsc in descs:
            dsc.wait()
        pltpu.sync_copy(rows_v, xsrc_hbm.at[pl.ds(rel * SL, CHUNK)])
        return carry

    lax.fori_loop(0, NCH, body, 0)


def _scatter_body(half, dst_hbm, msg_hbm, part_hbm, idx_v, rows_v, zbuf, acc):
    c = lax.axis_index("c")
    s = lax.axis_index("s")

    def zrow(i, carry):
        zbuf[i, :] = jnp.zeros((D,), jnp.float32)
        return carry

    lax.fori_loop(0, TPA // 16, zrow, 0)

    def zcopy(m, carry):
        pltpu.sync_copy(zbuf, acc.at[pl.ds(s * TPA + m * (TPA // 16), TPA // 16)])
        return carry

    lax.fori_loop(0, 16, zcopy, 0)
    plsc.subcore_barrier()

    wid = c * NS + s

    def body(k, carry):
        rel = wid * ROWS_PER_W + k * SLC
        pltpu.sync_copy(dst_hbm.at[pl.ds(half * HALF_ROWS + rel, SLC)], idx_v)
        pltpu.sync_copy(msg_hbm.at[pl.ds(rel * SL, CHUNK)], rows_v)
        for j in range(SLC):
            pltpu.sync_copy(rows_v.at[pl.ds(j * SL, SL)],
                            acc.at[idx_v.at[j]], add=True)
        return carry

    lax.fori_loop(0, NCH, body, 0)

    plsc.subcore_barrier()
    pltpu.sync_copy(acc.at[pl.ds(s * TPA, TPA)],
                    part_hbm.at[c].at[pl.ds(s * TPA, TPA)])


TEW = 512


def _msg_body(xw_ref, eat_ref, r_ref, s_ref, w2_ref, b_ref, out_ref):
    xw = xw_ref[...].astype(jnp.bfloat16)
    eat = eat_ref[...].astype(jnp.bfloat16)
    ea_n = jax.lax.dot_general(eat, s_ref[...], (((0,), (0,)), ((), ())),
                               preferred_element_type=jnp.float32)
    ar = ea_n.reshape(TEW, 2048)
    xr = jnp.dot(xw, r_ref[...], preferred_element_type=jnp.float32)
    t = (xr * ar).astype(jnp.bfloat16)
    out_ref[...] = (jnp.dot(t, w2_ref[...], preferred_element_type=jnp.float32)
                    + jnp.dot(xw, b_ref[...], preferred_element_type=jnp.float32))


TNW = 392


def _comb_body(pa_ref, pb_ref, x_ref, rwt_ref, rb_ref, out_ref):
    root = jnp.dot(x_ref[...].astype(jnp.bfloat16), rwt_ref[...],
                   preferred_element_type=jnp.float32)
    out_ref[...] = (pa_ref[0] + pa_ref[1] + pb_ref[0] + pb_ref[1]
                    + root + rb_ref[0:1, :])


def kernel(x, edge_index, edge_attr, root_w, root_b, edge_w, edge_b):
    x = x.astype(jnp.float32)
    src = edge_index[0].astype(jnp.int32)
    dst = edge_index[1].astype(jnp.int32)
    pad = E_PAD - E
    pad_ids = jnp.arange(pad, dtype=jnp.int32)
    src_p = jnp.concatenate([src, pad_ids % N_NODES])
    dst_p = jnp.concatenate([dst, N_NODES + pad_ids % (ACC_N - N_NODES)])
    src2d = src_p.reshape(E_PAD // SL, SL)
    dst2d = dst_p.reshape(E_PAD // SL, SL)

    mesh = plsc.VectorSubcoreMesh(core_axis_name="c", subcore_axis_name="s",
                                  num_cores=NC, num_subcores=NS)
    sc_params = pltpu.CompilerParams(use_tc_tiling_on_sc=False)

    def make_gather(half):
        return pl.kernel(
            lambda *a: _gather_body(half, *a),
            out_type=jax.ShapeDtypeStruct((E_HALF, D), jnp.float32),
            mesh=mesh,
            compiler_params=sc_params,
            scratch_types=[
                pltpu.VMEM((SLC, SL), jnp.int32),
                pltpu.VMEM((CHUNK, D), jnp.float32),
                pltpu.SemaphoreType.DMA,
            ],
        )

    def make_scatter(half):
        return pl.kernel(
            lambda *a: _scatter_body(half, *a),
            out_type=jax.ShapeDtypeStruct((NC, ACC_N, D), jnp.float32),
            mesh=mesh,
            compiler_params=sc_params,
            scratch_types=[
                pltpu.VMEM((SLC, SL), jnp.int32),
                pltpu.VMEM((CHUNK, D), jnp.float32),
                pltpu.VMEM((TPA // 16, D), jnp.float32),
                pltpu.VMEM_SHARED((ACC_N, D), jnp.float32),
            ],
        )

    eat = edge_attr.T

    eye = jnp.eye(D, dtype=jnp.float32)
    eye8 = jnp.eye(8, dtype=jnp.float32)
    r_mat = jnp.kron(eye, jnp.ones((1, D), jnp.float32))
    s_mat = jnp.tile(eye, (1, D))
    w2 = edge_w.reshape(D, D, D).transpose(1, 2, 0).reshape(D * D, D)
    b_mat = edge_b.reshape(D, D).T
    bd_r = jnp.kron(eye8, r_mat).astype(jnp.bfloat16)
    bd_w2 = jnp.kron(eye8, w2).astype(jnp.bfloat16)
    bd_b = jnp.kron(eye8, b_mat).astype(jnp.bfloat16)
    s_bf = s_mat.astype(jnp.bfloat16)

    def msg_half(xw_h, half):
        base = half * (E_HALF // 8 // TEW)
        return pl.pallas_call(
            _msg_body,
            grid=(E_HALF // 8 // TEW,),
            in_specs=[
                pl.BlockSpec((TEW, 128), lambda i: (i, 0)),
                pl.BlockSpec((D, TEW * 8), lambda i, b=base: (0, i + b)),
                pl.BlockSpec((128, 2048), lambda i: (0, 0)),
                pl.BlockSpec((D, D * D), lambda i: (0, 0)),
                pl.BlockSpec((2048, 128), lambda i: (0, 0)),
                pl.BlockSpec((128, 128), lambda i: (0, 0)),
            ],
            out_specs=pl.BlockSpec((TEW, 128), lambda i: (i, 0)),
            out_shape=jax.ShapeDtypeStruct((E_HALF // 8, 128), jnp.float32),
        )(xw_h, eat, bd_r, s_bf, bd_w2, bd_b)

    x_src0 = make_gather(0)(x, src2d)
    x_src1 = make_gather(1)(x, src2d)
    msg0 = msg_half(x_src0.reshape(E_HALF // 8, 128), 0)
    msg1 = msg_half(x_src1.reshape(E_HALF // 8, 128), 1)
    parts0 = make_scatter(0)(dst2d, msg0.reshape(E_HALF, D))
    parts1 = make_scatter(1)(dst2d, msg1.reshape(E_HALF, D))

    parts0_w = parts0.reshape(NC, ACC_N * D // 128, 128)
    parts1_w = parts1.reshape(NC, ACC_N * D // 128, 128)
    xw8 = x.reshape(N_NODES * D // 128, 128)
    bd_rwt = jnp.kron(eye8, root_w.T).astype(jnp.bfloat16)
    rbw = jnp.broadcast_to(jnp.tile(root_b, 8), (8, 128))
    acc_w = ACC_N * D // 128
    out_w = pl.pallas_call(
        _comb_body,
        grid=(acc_w // TNW,),
        in_specs=[
            pl.BlockSpec((NC, TNW, 128), lambda i: (0, i, 0)),
            pl.BlockSpec((NC, TNW, 128), lambda i: (0, i, 0)),
            pl.BlockSpec((TNW, 128), lambda i: (i, 0)),
            pl.BlockSpec((128, 128), lambda i: (0, 0)),
            pl.BlockSpec((8, 128), lambda i: (0, 0)),
        ],
        out_specs=pl.BlockSpec((TNW, 128), lambda i: (i, 0)),
        out_shape=jax.ShapeDtypeStruct((acc_w, 128), jnp.float32),
    )(parts0_w, parts1_w, xw8, bd_rwt, rbw)
    return out_w.reshape(ACC_N, D)[:N_NODES]
